# baseline (speedup 1.0000x reference)
import jax
import jax.numpy as jnp
from jax.experimental import pallas as pl
from jax.experimental.pallas import tpu as pltpu

EPS_ = 1e-09
NGATES = 64
BT = 1024
NK = 4


def _gating_block(x_ref, w_ref, o_ref, acc_ref):
    k = pl.program_id(1)
    part = jnp.dot(x_ref[...], w_ref[...], preferred_element_type=jnp.float32)

    @pl.when(k == 0)
    def _():
        acc_ref[...] = part

    @pl.when(k > 0)
    def _():
        acc_ref[...] += part

    @pl.when(k == NK - 1)
    def _():
        logits = acc_ref[...]
        m = jnp.max(logits, axis=-1, keepdims=True)
        e = jnp.exp(logits - m)
        s = jnp.sum(e, axis=-1, keepdims=True)
        eq1 = e == 1.0
        e2 = jnp.where(eq1, 0.0, e)
        em2 = jnp.max(e2, axis=-1, keepdims=True)
        v1 = 1.0 / s
        v2 = em2 / s
        denom = v1 + v2 + EPS_
        eq2 = (e2 == em2) & ~eq1
        out = jnp.where(eq1, v1 / denom, 0.0)
        out = jnp.where(eq2, v2 / denom, out)
        cols0 = jax.lax.broadcasted_iota(jnp.int32, out.shape, 1) == 0
        o_ref[...] = jnp.where(cols0 & (v2 == 0.0), 0.0, out)


def kernel(x, w_gating):
    b, group, dim = x.shape
    n = b * group
    bk = dim // NK
    x2 = x.reshape(n, dim)
    grid = (n // BT, NK)
    out = pl.pallas_call(
        _gating_block,
        grid=grid,
        in_specs=[
            pl.BlockSpec((BT, bk), lambda i, k: (i, k)),
            pl.BlockSpec((bk, NGATES), lambda i, k: (k, 0)),
        ],
        out_specs=pl.BlockSpec((BT, NGATES), lambda i, k: (i, 0)),
        out_shape=jax.ShapeDtypeStruct((n, NGATES), jnp.float32),
        scratch_shapes=[pltpu.VMEM((BT, NGATES), jnp.float32)],
        compiler_params=pltpu.CompilerParams(
            dimension_semantics=("parallel", "arbitrary"),
        ),
    )(x2, w_gating)
    return out.reshape(b, group, NGATES)


# speedup vs baseline: 1.3441x; 1.3441x over previous
import jax
import jax.numpy as jnp
from jax.experimental import pallas as pl
from jax.experimental.pallas import tpu as pltpu

EPS_ = 1e-09
NGATES = 64
BT = 1024


def _gating_block(x_ref, w_ref, o_ref):
    logits = jnp.dot(x_ref[...], w_ref[...], preferred_element_type=jnp.float32)
    m = jnp.max(logits, axis=-1, keepdims=True)
    e = jnp.exp(logits - m)
    s = jnp.sum(e, axis=-1, keepdims=True)
    eq1 = e == 1.0
    e2 = jnp.where(eq1, 0.0, e)
    em2 = jnp.max(e2, axis=-1, keepdims=True)
    v1 = 1.0 / s
    v2 = em2 / s
    denom = v1 + v2 + EPS_
    eq2 = (e2 == em2) & ~eq1
    out = jnp.where(eq1, v1 / denom, 0.0)
    out = jnp.where(eq2, v2 / denom, out)
    cols0 = jax.lax.broadcasted_iota(jnp.int32, out.shape, 1) == 0
    out = jnp.where(cols0 & (v2 == 0.0), 0.0, out)
    o_ref[...] = out


def kernel(x, w_gating):
    b, group, dim = x.shape
    n = b * group
    x2 = x.reshape(n, dim)
    grid = (n // BT,)
    out = pl.pallas_call(
        _gating_block,
        grid=grid,
        in_specs=[
            pl.BlockSpec((BT, dim), lambda i: (i, 0)),
            pl.BlockSpec((dim, NGATES), lambda i: (0, 0)),
        ],
        out_specs=pl.BlockSpec((BT, NGATES), lambda i: (i, 0)),
        out_shape=jax.ShapeDtypeStruct((n, NGATES), jnp.float32),
        compiler_params=pltpu.CompilerParams(
            dimension_semantics=("parallel",),
        ),
    )(x2, w_gating)
    return out.reshape(b, group, NGATES)
